# baseline (speedup 1.0000x reference)
import functools

import jax
import jax.numpy as jnp
from jax import lax
from jax.experimental import pallas as pl
from jax.experimental.pallas import tpu as pltpu, tpu_sc as plsc

N = 10000
H = 128
E = 320000
LIB = 1001
NC_SC = 2
NS = 16
NW = NC_SC * NS

XROWS = 80
XWORK = 2 * (XROWS // 8)

EROWS = 2560
EPW = EROWS // NW
NBLK = EPW // 8
AGG_ROWS = N + NS

OSLAB = 624
ZLAST = AGG_ROWS - 15 * OSLAB


@functools.cache
def _emb_gather():
    mesh = plsc.VectorSubcoreMesh(core_axis_name="c", subcore_axis_name="s",
                                  num_cores=NC_SC, num_subcores=NS)
    return pl.kernel(
        _emb_gather_body,
        out_type=jax.ShapeDtypeStruct((XROWS * 128, H), jnp.float32),
        mesh=mesh,
        scratch_types=[
            pltpu.VMEM((8, 128), jnp.int32),
            pltpu.VMEM((512, H), jnp.float32),
            pltpu.SemaphoreType.DMA,
        ],
    )


def _emb_gather_body(emb_hbm, x_hbm, out_hbm, idx_v, rows_v, sem):
    c = lax.axis_index("c")
    s = lax.axis_index("s")
    wid = s * NC_SC + c

    @pl.when(wid < XWORK)
    def _():
        chunk = wid // 2
        half = wid % 2
        pltpu.sync_copy(x_hbm.at[pl.ds(chunk * 8, 8)], idx_v)
        gs = []
        for j in range(4):
            gs.append(pltpu.async_copy(
                emb_hbm.at[idx_v.at[4 * half + j]],
                rows_v.at[pl.ds(j * 128, 128)], sem))
        for g in gs:
            g.wait()
        pltpu.sync_copy(
            rows_v, out_hbm.at[pl.ds(chunk * 1024 + half * 512, 512)])


@functools.cache
def _agg_scatter(h_rows):
    mesh = plsc.VectorSubcoreMesh(core_axis_name="c", subcore_axis_name="s",
                                  num_cores=NC_SC, num_subcores=NS)
    return pl.kernel(
        _agg_scatter_body,
        out_type=jax.ShapeDtypeStruct((2 * N, H), jnp.float32),
        mesh=mesh,
        scratch_types=[
            pltpu.VMEM((8, 128), jnp.int32),
            pltpu.VMEM((8, 128), jnp.int32),
            pltpu.VMEM((8, 128), jnp.int32),
            pltpu.VMEM((8, 128), jnp.int32),
            pltpu.VMEM((128, H), jnp.float32),
            pltpu.VMEM((128, H), jnp.float32),
            pltpu.VMEM_SHARED((AGG_ROWS, H), jnp.float32),
            pltpu.SemaphoreType.DMA,
            pltpu.SemaphoreType.DMA,
            pltpu.SemaphoreType.DMA,
        ],
    )


def _agg_scatter_body(h_hbm, src_hbm, dst_hbm, zero_hbm, out_hbm,
                      src_v0, dst_v0, src_v1, dst_v1, buf_a, buf_b, agg_sh,
                      semg, sems, semi):
    c = lax.axis_index("c")
    s = lax.axis_index("s")
    bufs = (buf_a, buf_b)
    idxs = ((src_v0, dst_v0), (src_v1, dst_v1))

    base = c * (EROWS // NC_SC) + s * EPW

    def idx_load(pair, r0):
        pltpu.async_copy(src_hbm.at[pl.ds(r0, 8)], idxs[pair][0], semi)
        pltpu.async_copy(dst_hbm.at[pl.ds(r0, 8)], idxs[pair][1], semi)

    def idx_wait(pair):
        pltpu.make_async_copy(src_hbm.at[pl.ds(base, 8)], idxs[pair][0],
                              semi).wait()
        pltpu.make_async_copy(dst_hbm.at[pl.ds(base, 8)], idxs[pair][1],
                              semi).wait()

    def g0_issue(pair):
        pltpu.async_copy(h_hbm.at[idxs[pair][0].at[0]], bufs[0], semg)

    def g0_wait(pair):
        pltpu.make_async_copy(h_hbm.at[idxs[pair][0].at[0]], bufs[0],
                              semg).wait()

    idx_load(0, base)
    idx_load(1, base + 8)
    idx_wait(0)
    g0_issue(0)

    @pl.when(s < 15)
    def _():
        pltpu.sync_copy(zero_hbm.at[pl.ds(s * OSLAB, OSLAB)],
                        agg_sh.at[pl.ds(s * OSLAB, OSLAB)])

    @pl.when(s == 15)
    def _():
        pltpu.sync_copy(zero_hbm.at[pl.ds(15 * OSLAB, ZLAST)],
                        agg_sh.at[pl.ds(15 * OSLAB, ZLAST)])

    plsc.subcore_barrier()

    def process(pair, prefetch_r0, has_next):
        src_v, dst_v = idxs[pair]
        g = [None] * 8
        sc = [None] * 8
        for j in range(8):
            if j + 1 < 8:
                if j >= 1:
                    sc[j - 1].wait()
                g[j + 1] = pltpu.async_copy(h_hbm.at[src_v.at[j + 1]],
                                            bufs[(j + 1) % 2], semg)
            if j == 0:
                g0_wait(pair)
            else:
                g[j].wait()
            sc[j] = pltpu.async_copy(bufs[j % 2], agg_sh.at[dst_v.at[j]],
                                     sems, add=True)
        sc[6].wait()
        if has_next is not None:
            @pl.when(has_next)
            def _():
                idx_wait(1 - pair)
                g0_issue(1 - pair)
        else:
            idx_wait(1 - pair)
            g0_issue(1 - pair)
        sc[7].wait()
        idx_load(pair, prefetch_r0)

    def body(gr, carry):
        process(0, base + 8 * jnp.minimum(2 * gr + 2, NBLK - 1), None)
        process(1, base + 8 * jnp.minimum(2 * gr + 3, NBLK - 1), gr < 4)
        return carry

    lax.fori_loop(0, NBLK // 2, body, 0)
    idx_wait(0)
    idx_wait(1)
    plsc.subcore_barrier()

    @pl.when(s < 15)
    def _():
        pltpu.sync_copy(agg_sh.at[pl.ds(s * OSLAB, OSLAB)],
                        out_hbm.at[pl.ds(c * N + s * OSLAB, OSLAB)])

    @pl.when(s == 15)
    def _():
        pltpu.sync_copy(agg_sh.at[pl.ds(15 * OSLAB, N - 15 * OSLAB)],
                        out_hbm.at[pl.ds(c * N + 15 * OSLAB, N - 15 * OSLAB)])


@functools.cache
def _gin_dense(h_rows):
    def body(h_ref, part_ref, w1_ref, b1_ref, g1_ref, be1_ref,
             w2_ref, b2_ref, o_ref):
        h = h_ref[pl.ds(0, N), :] if h_rows > N else h_ref[...]
        z = h + part_ref[pl.ds(0, N), :] + part_ref[pl.ds(N, N), :]
        z = (jnp.dot(z, w1_ref[...], preferred_element_type=jnp.float32)
             + b1_ref[...])
        z = jnp.maximum(z, 0.0)
        mu = jnp.mean(z, axis=0, keepdims=True)
        var = jnp.mean((z - mu) ** 2, axis=0, keepdims=True)
        z = (z - mu) / jnp.sqrt(var + 1e-5) * g1_ref[...] + be1_ref[...]
        o_ref[...] = (jnp.dot(z, w2_ref[...],
                              preferred_element_type=jnp.float32)
                      + b2_ref[...])

    return pl.pallas_call(
        body, out_shape=jax.ShapeDtypeStruct((N, H), jnp.float32))


def _softplus(x):
    return jnp.maximum(x, 0.0) + jnp.log1p(jnp.exp(-jnp.abs(x)))


def _heads_body(h_ref, w1_ref, b1_ref, g1_ref, be1_ref, w2_ref, b2_ref,
                g2_ref, be2_ref, w3_ref, b3_ref, *o_ref):
    h = h_ref[...]
    z = jnp.dot(h, w1_ref[...], preferred_element_type=jnp.float32) + b1_ref[...]
    m = jnp.mean(z, axis=0, keepdims=True)
    v = jnp.mean((z - m) ** 2, axis=0, keepdims=True)
    z = (z - m) / jnp.sqrt(v + 1e-5) * g1_ref[...] + be1_ref[...]
    z = jnp.maximum(z, 0.0)
    z = jnp.dot(z, w2_ref[...], preferred_element_type=jnp.float32) + b2_ref[...]
    m = jnp.mean(z, axis=0, keepdims=True)
    v = jnp.mean((z - m) ** 2, axis=0, keepdims=True)
    z = (z - m) / jnp.sqrt(v + 1e-5) * g2_ref[...] + be2_ref[...]
    t = jnp.dot(z, w3_ref[...], preferred_element_type=jnp.float32) + b3_ref[...]
    for k, o in enumerate(o_ref):
        sec = t[:, 16 * k:16 * k + 10]
        if k % 3 == 1:
            sec = jnp.tanh(sec) * 180.0
        elif k % 3 == 2:
            sec = _softplus(sec)
        o[...] = sec


_heads = pl.pallas_call(
    _heads_body,
    out_shape=tuple(jax.ShapeDtypeStruct((N, 10), jnp.float32)
                    for _ in range(6)),
)


def _row(v):
    return v.reshape(1, -1)


def _heads_weights(pa, pb):
    cat = lambda k: jnp.concatenate([pa[k], pb[k]], axis=-1)
    z64 = jnp.zeros((64, 64), jnp.float32)
    w2 = jnp.concatenate([
        jnp.concatenate([pa['W2'], z64], axis=1),
        jnp.concatenate([z64, pb['W2']], axis=1)], axis=0)
    pad6 = lambda m: jnp.pad(m, ((0, 0), (0, 6)))
    w3a = jnp.concatenate([pad6(pa['Ww']), pad6(pa['Wm']), pad6(pa['Wk'])],
                          axis=1)
    w3b = jnp.concatenate([pad6(pb['Ww']), pad6(pb['Wm']), pad6(pb['Wk'])],
                          axis=1)
    z48 = jnp.zeros((64, 48), jnp.float32)
    w3 = jnp.concatenate([
        jnp.concatenate([w3a, z48], axis=1),
        jnp.concatenate([z48, w3b], axis=1)], axis=0)
    padb = lambda v: jnp.pad(v, (0, 6))
    b3 = jnp.concatenate([padb(pa['bw']), padb(pa['bm']), padb(pa['bk']),
                          padb(pb['bw']), padb(pb['bm']), padb(pb['bk'])])
    return (cat('W1'), _row(cat('b1')), _row(cat('g1')), _row(cat('be1')),
            w2, _row(cat('b2')), _row(cat('g2')), _row(cat('be2')),
            w3, _row(b3))


def kernel(x, edge_index, params):
    x = x.astype(jnp.int32)
    src = edge_index[0].astype(jnp.int32)
    dst = edge_index[1].astype(jnp.int32)

    xpad = jnp.arange(XROWS * 128 - N, dtype=jnp.int32) % LIB
    x2d = jnp.concatenate([x, xpad]).reshape(XROWS, 128)
    h = _emb_gather()(params['emb'], x2d)

    epad = jnp.arange(EROWS * 128 - E, dtype=jnp.int32)
    src_p = jnp.concatenate([src, (epad * 97) % N]).reshape(EROWS, 128)
    dst_p = jnp.concatenate([dst, N + (epad % NS)]).reshape(EROWS, 128)
    zeros = jnp.zeros((AGG_ROWS, H), jnp.float32)

    for p in params['gin']:
        part = _agg_scatter(h.shape[0])(h, src_p, dst_p, zeros)
        h = _gin_dense(h.shape[0])(h, part, p['W1'], _row(p['b1']),
                                   _row(p['g1']), _row(p['be1']), p['W2'],
                                   _row(p['b2']))

    outs = _heads(h, *_heads_weights(params['head_vm'], params['head_g']))
    one = params['emb'][0, 0] * 0.0 + 1.0
    return tuple((o * one).reshape(N, 2, 5) for o in outs)


# speedup vs baseline: 1.0474x; 1.0474x over previous
import functools

import jax
import jax.numpy as jnp
from jax import lax
from jax.experimental import pallas as pl
from jax.experimental.pallas import tpu as pltpu, tpu_sc as plsc

N = 10000
H = 128
E = 320000
LIB = 1001
NC_SC = 2
NS = 16
NW = NC_SC * NS

XROWS = 80
XWORK = 2 * (XROWS // 8)

EROWS = 2560
EPW = EROWS // NW
NBLK = EPW // 8
AGG_ROWS = N + NS

OSLAB = 624
ZLAST = AGG_ROWS - 15 * OSLAB


@functools.cache
def _emb_gather():
    mesh = plsc.VectorSubcoreMesh(core_axis_name="c", subcore_axis_name="s",
                                  num_cores=NC_SC, num_subcores=NS)
    return pl.kernel(
        _emb_gather_body,
        out_type=jax.ShapeDtypeStruct((XROWS * 128, H), jnp.float32),
        mesh=mesh,
        scratch_types=[
            pltpu.VMEM((8, 128), jnp.int32),
            pltpu.VMEM((512, H), jnp.float32),
            pltpu.SemaphoreType.DMA,
        ],
    )


def _emb_gather_body(emb_hbm, x_hbm, out_hbm, idx_v, rows_v, sem):
    c = lax.axis_index("c")
    s = lax.axis_index("s")
    wid = s * NC_SC + c

    @pl.when(wid < XWORK)
    def _():
        chunk = wid // 2
        half = wid % 2
        pltpu.sync_copy(x_hbm.at[pl.ds(chunk * 8, 8)], idx_v)
        gs = []
        for j in range(4):
            gs.append(pltpu.async_copy(
                emb_hbm.at[idx_v.at[4 * half + j]],
                rows_v.at[pl.ds(j * 128, 128)], sem))
        for g in gs:
            g.wait()
        pltpu.sync_copy(
            rows_v, out_hbm.at[pl.ds(chunk * 1024 + half * 512, 512)])


@functools.cache
def _agg_scatter(h_rows):
    mesh = plsc.VectorSubcoreMesh(core_axis_name="c", subcore_axis_name="s",
                                  num_cores=NC_SC, num_subcores=NS)
    return pl.kernel(
        _agg_scatter_body,
        out_type=jax.ShapeDtypeStruct((2 * N, H), jnp.float32),
        mesh=mesh,
        scratch_types=[
            pltpu.VMEM((8, 128), jnp.int32),
            pltpu.VMEM((8, 128), jnp.int32),
            pltpu.VMEM((8, 128), jnp.int32),
            pltpu.VMEM((8, 128), jnp.int32),
            pltpu.VMEM((128, H), jnp.float32),
            pltpu.VMEM((128, H), jnp.float32),
            pltpu.VMEM_SHARED((AGG_ROWS, H), jnp.float32),
            pltpu.SemaphoreType.DMA,
            pltpu.SemaphoreType.DMA,
            pltpu.SemaphoreType.DMA,
        ],
    )


def _agg_scatter_body(h_hbm, src_hbm, dst_hbm, zero_hbm, out_hbm,
                      src_v0, dst_v0, src_v1, dst_v1, buf_a, buf_b, agg_sh,
                      semg, sems, semi):
    c = lax.axis_index("c")
    s = lax.axis_index("s")
    bufs = (buf_a, buf_b)
    idxs = ((src_v0, dst_v0), (src_v1, dst_v1))

    base = c * (EROWS // NC_SC) + s * EPW

    def idx_load(pair, r0):
        pltpu.async_copy(src_hbm.at[pl.ds(r0, 8)], idxs[pair][0], semi)
        pltpu.async_copy(dst_hbm.at[pl.ds(r0, 8)], idxs[pair][1], semi)

    def idx_wait(pair):
        pltpu.make_async_copy(src_hbm.at[pl.ds(base, 8)], idxs[pair][0],
                              semi).wait()
        pltpu.make_async_copy(dst_hbm.at[pl.ds(base, 8)], idxs[pair][1],
                              semi).wait()

    def g0_issue(pair):
        pltpu.async_copy(h_hbm.at[idxs[pair][0].at[0]], bufs[0], semg)

    def g0_wait(pair):
        pltpu.make_async_copy(h_hbm.at[idxs[pair][0].at[0]], bufs[0],
                              semg).wait()

    idx_load(0, base)
    idx_load(1, base + 8)
    idx_wait(0)
    g0_issue(0)

    @pl.when(s < 15)
    def _():
        pltpu.sync_copy(zero_hbm.at[pl.ds(s * OSLAB, OSLAB)],
                        agg_sh.at[pl.ds(s * OSLAB, OSLAB)])

    @pl.when(s == 15)
    def _():
        pltpu.sync_copy(zero_hbm.at[pl.ds(15 * OSLAB, ZLAST)],
                        agg_sh.at[pl.ds(15 * OSLAB, ZLAST)])

    plsc.subcore_barrier()

    def process(pair, prefetch_r0, has_next):
        src_v, dst_v = idxs[pair]
        g = [None] * 8
        sc = [None] * 8
        for j in range(8):
            if j + 1 < 8:
                if j >= 1:
                    sc[j - 1].wait()
                g[j + 1] = pltpu.async_copy(h_hbm.at[src_v.at[j + 1]],
                                            bufs[(j + 1) % 2], semg)
            if j == 0:
                g0_wait(pair)
            else:
                g[j].wait()
            sc[j] = pltpu.async_copy(bufs[j % 2], agg_sh.at[dst_v.at[j]],
                                     sems, add=True)
        sc[6].wait()
        if has_next is not None:
            @pl.when(has_next)
            def _():
                idx_wait(1 - pair)
                g0_issue(1 - pair)
        else:
            idx_wait(1 - pair)
            g0_issue(1 - pair)
        sc[7].wait()
        idx_load(pair, prefetch_r0)

    def body(gr, carry):
        process(0, base + 8 * jnp.minimum(2 * gr + 2, NBLK - 1), None)
        process(1, base + 8 * jnp.minimum(2 * gr + 3, NBLK - 1), gr < 4)
        return carry

    lax.fori_loop(0, NBLK // 2, body, 0)
    idx_wait(0)
    idx_wait(1)
    plsc.subcore_barrier()

    @pl.when(s < 15)
    def _():
        pltpu.sync_copy(agg_sh.at[pl.ds(s * OSLAB, OSLAB)],
                        out_hbm.at[pl.ds(c * N + s * OSLAB, OSLAB)])

    @pl.when(s == 15)
    def _():
        pltpu.sync_copy(agg_sh.at[pl.ds(15 * OSLAB, N - 15 * OSLAB)],
                        out_hbm.at[pl.ds(c * N + 15 * OSLAB, N - 15 * OSLAB)])


@functools.cache
def _gin_dense(h_rows):
    def body(h_ref, part_ref, w1_ref, b1_ref, g1_ref, be1_ref,
             w2_ref, b2_ref, o_ref):
        h = h_ref[pl.ds(0, N), :] if h_rows > N else h_ref[...]
        z = h + part_ref[pl.ds(0, N), :] + part_ref[pl.ds(N, N), :]
        z = (jnp.dot(z, w1_ref[...], preferred_element_type=jnp.float32)
             + b1_ref[...])
        z = jnp.maximum(z, 0.0)
        mu = jnp.mean(z, axis=0, keepdims=True)
        var = jnp.mean((z - mu) ** 2, axis=0, keepdims=True)
        z = (z - mu) / jnp.sqrt(var + 1e-5) * g1_ref[...] + be1_ref[...]
        o_ref[...] = (jnp.dot(z, w2_ref[...],
                              preferred_element_type=jnp.float32)
                      + b2_ref[...])

    return pl.pallas_call(
        body, out_shape=jax.ShapeDtypeStruct((N, H), jnp.float32))


def _softplus(x):
    return jnp.maximum(x, 0.0) + jnp.log1p(jnp.exp(-jnp.abs(x)))


def _heads_body(h_ref, part_ref, gw1_ref, gb1_ref, gg1_ref, gbe1_ref,
                gw2_ref, gb2_ref, w1_ref, b1_ref, g1_ref, be1_ref, w2_ref,
                b2_ref, g2_ref, be2_ref, w3_ref, b3_ref, *o_ref):
    hz = (h_ref[...] + part_ref[pl.ds(0, N), :] + part_ref[pl.ds(N, N), :])
    hz = (jnp.dot(hz, gw1_ref[...], preferred_element_type=jnp.float32)
          + gb1_ref[...])
    hz = jnp.maximum(hz, 0.0)
    gm = jnp.mean(hz, axis=0, keepdims=True)
    gv = jnp.mean((hz - gm) ** 2, axis=0, keepdims=True)
    hz = (hz - gm) / jnp.sqrt(gv + 1e-5) * gg1_ref[...] + gbe1_ref[...]
    h = (jnp.dot(hz, gw2_ref[...], preferred_element_type=jnp.float32)
         + gb2_ref[...])
    z = jnp.dot(h, w1_ref[...], preferred_element_type=jnp.float32) + b1_ref[...]
    m = jnp.mean(z, axis=0, keepdims=True)
    v = jnp.mean((z - m) ** 2, axis=0, keepdims=True)
    z = (z - m) / jnp.sqrt(v + 1e-5) * g1_ref[...] + be1_ref[...]
    z = jnp.maximum(z, 0.0)
    z = jnp.dot(z, w2_ref[...], preferred_element_type=jnp.float32) + b2_ref[...]
    m = jnp.mean(z, axis=0, keepdims=True)
    v = jnp.mean((z - m) ** 2, axis=0, keepdims=True)
    z = (z - m) / jnp.sqrt(v + 1e-5) * g2_ref[...] + be2_ref[...]
    t = jnp.dot(z, w3_ref[...], preferred_element_type=jnp.float32) + b3_ref[...]
    for k, o in enumerate(o_ref):
        sec = t[:, 16 * k:16 * k + 10]
        if k % 3 == 1:
            sec = jnp.tanh(sec) * 180.0
        elif k % 3 == 2:
            sec = _softplus(sec)
        o[...] = sec


_heads = pl.pallas_call(
    _heads_body,
    out_shape=tuple(jax.ShapeDtypeStruct((N, 10), jnp.float32)
                    for _ in range(6)),
)


def _row(v):
    return v.reshape(1, -1)


def _heads_weights(pa, pb):
    cat = lambda k: jnp.concatenate([pa[k], pb[k]], axis=-1)
    z64 = jnp.zeros((64, 64), jnp.float32)
    w2 = jnp.concatenate([
        jnp.concatenate([pa['W2'], z64], axis=1),
        jnp.concatenate([z64, pb['W2']], axis=1)], axis=0)
    pad6 = lambda m: jnp.pad(m, ((0, 0), (0, 6)))
    w3a = jnp.concatenate([pad6(pa['Ww']), pad6(pa['Wm']), pad6(pa['Wk'])],
                          axis=1)
    w3b = jnp.concatenate([pad6(pb['Ww']), pad6(pb['Wm']), pad6(pb['Wk'])],
                          axis=1)
    z48 = jnp.zeros((64, 48), jnp.float32)
    w3 = jnp.concatenate([
        jnp.concatenate([w3a, z48], axis=1),
        jnp.concatenate([z48, w3b], axis=1)], axis=0)
    padb = lambda v: jnp.pad(v, (0, 6))
    b3 = jnp.concatenate([padb(pa['bw']), padb(pa['bm']), padb(pa['bk']),
                          padb(pb['bw']), padb(pb['bm']), padb(pb['bk'])])
    return (cat('W1'), _row(cat('b1')), _row(cat('g1')), _row(cat('be1')),
            w2, _row(cat('b2')), _row(cat('g2')), _row(cat('be2')),
            w3, _row(b3))


def kernel(x, edge_index, params):
    x = x.astype(jnp.int32)
    src = edge_index[0].astype(jnp.int32)
    dst = edge_index[1].astype(jnp.int32)

    xpad = jnp.arange(XROWS * 128 - N, dtype=jnp.int32) % LIB
    x2d = jnp.concatenate([x, xpad]).reshape(XROWS, 128)
    h = _emb_gather()(params['emb'], x2d)

    epad = jnp.arange(EROWS * 128 - E, dtype=jnp.int32)
    src_p = jnp.concatenate([src, (epad * 97) % N]).reshape(EROWS, 128)
    dst_p = jnp.concatenate([dst, N + (epad % NS)]).reshape(EROWS, 128)
    zeros = jnp.zeros((AGG_ROWS, H), jnp.float32)

    for p in params['gin'][:2]:
        part = _agg_scatter(h.shape[0])(h, src_p, dst_p, zeros)
        h = _gin_dense(h.shape[0])(h, part, p['W1'], _row(p['b1']),
                                   _row(p['g1']), _row(p['be1']), p['W2'],
                                   _row(p['b2']))

    part = _agg_scatter(h.shape[0])(h, src_p, dst_p, zeros)
    p3 = params['gin'][2]
    outs = _heads(h, part, p3['W1'], _row(p3['b1']), _row(p3['g1']),
                  _row(p3['be1']), p3['W2'], _row(p3['b2']),
                  *_heads_weights(params['head_vm'], params['head_g']))
    return tuple(o.reshape(N, 2, 5) for o in outs)
